# baseline (speedup 1.0000x reference)
import functools

import jax
import jax.numpy as jnp
from jax import lax
from jax.experimental import pallas as pl
from jax.experimental.pallas import tpu as pltpu
from jax.experimental.pallas import tpu_sc as plsc

SEQ = 200
EMBED_DIM = 64
PAD_B = 256

_info = plsc.get_sparse_core_info()
_NC, _NS = _info.num_cores, _info.num_subcores
_NW = _NC * _NS
_B_PER_W = PAD_B // _NW

_sc_mesh = plsc.VectorSubcoreMesh(core_axis_name="c", subcore_axis_name="s")


@functools.partial(
    pl.kernel,
    mesh=_sc_mesh,
    out_type=jax.ShapeDtypeStruct((PAD_B, EMBED_DIM), jnp.float32),
    scratch_types=[
        pltpu.VMEM((_B_PER_W,), jnp.int32),
        pltpu.VMEM((_B_PER_W, EMBED_DIM), jnp.float32),
        pltpu.SemaphoreType.DMA,
    ],
    compiler_params=pltpu.CompilerParams(use_tc_tiling_on_sc=False),
)
def _sc_gather(table_hbm, idx_hbm, out_hbm, idx_v, rows_v, sem):
    wid = lax.axis_index("s") * _NC + lax.axis_index("c")
    base = wid * _B_PER_W
    pltpu.sync_copy(idx_hbm.at[pl.ds(base, _B_PER_W)], idx_v)
    pltpu.async_copy(table_hbm.at[idx_v], rows_v, sem).wait()
    pltpu.sync_copy(rows_v, out_hbm.at[pl.ds(base, _B_PER_W)])


def _tc_linear_body(g_ref, wt_ref, b_ref, o_ref):
    o_ref[...] = (
        jnp.dot(g_ref[...], wt_ref[...], preferred_element_type=jnp.float32)
        + b_ref[...]
    )


def kernel(sentence, emb_table, W, b):
    idx = sentence[-1].astype(jnp.int32)
    idx_pad = jnp.zeros((PAD_B,), jnp.int32).at[:SEQ].set(idx)
    rows = _sc_gather(emb_table, idx_pad)
    wt = jnp.zeros((EMBED_DIM, 128), jnp.float32).at[:, :2].set(W.T)
    bp = jnp.zeros((1, 128), jnp.float32).at[0, :2].set(b)
    y = pl.pallas_call(
        _tc_linear_body,
        out_shape=jax.ShapeDtypeStruct((PAD_B, 128), jnp.float32),
    )(rows, wt, bp)
    return y[:SEQ, :2]


# speedup vs baseline: 1.7050x; 1.7050x over previous
import functools

import jax
import jax.numpy as jnp
from jax import lax
from jax.experimental import pallas as pl
from jax.experimental.pallas import tpu as pltpu
from jax.experimental.pallas import tpu_sc as plsc

SEQ = 200
EMBED_DIM = 64
PAD_B = 256

_info = plsc.get_sparse_core_info()
_NC, _NS = _info.num_cores, _info.num_subcores
_NW = _NC * _NS
_B_PER_W = PAD_B // _NW

_sc_mesh = plsc.VectorSubcoreMesh(core_axis_name="c", subcore_axis_name="s")


@functools.partial(
    pl.kernel,
    mesh=_sc_mesh,
    out_type=jax.ShapeDtypeStruct((PAD_B, EMBED_DIM), jnp.float32),
    scratch_types=[
        pltpu.VMEM((16,), jnp.int32),
        pltpu.VMEM((_B_PER_W, EMBED_DIM), jnp.float32),
        pltpu.SemaphoreType.DMA,
    ],
    compiler_params=pltpu.CompilerParams(use_tc_tiling_on_sc=True),
)
def _sc_gather(table_hbm, idx_hbm, out_hbm, idx_v, rows_v, sem):
    wid = lax.axis_index("s") * _NC + lax.axis_index("c")
    base = wid * _B_PER_W
    pltpu.sync_copy(idx_hbm.at[pl.ds(base, _B_PER_W)], idx_v.at[pl.ds(0, _B_PER_W)])
    idx_vec = idx_v[...]
    copies = []
    for j in range(_B_PER_W):
        copies.append(
            pltpu.async_copy(
                table_hbm.at[pl.ds(idx_vec[j], 1)], rows_v.at[pl.ds(j, 1)], sem
            )
        )
    for c in copies:
        c.wait()
    pltpu.sync_copy(rows_v, out_hbm.at[pl.ds(base, _B_PER_W)])


def _tc_linear_body(g_ref, wt_ref, b_ref, o_ref):
    o_ref[...] = (
        jnp.dot(g_ref[...], wt_ref[...], preferred_element_type=jnp.float32)
        + b_ref[...]
    )


def kernel(sentence, emb_table, W, b):
    idx = sentence[-1].astype(jnp.int32)
    idx_pad = jnp.zeros((PAD_B,), jnp.int32).at[:SEQ].set(idx)
    rows = _sc_gather(emb_table, idx_pad)
    wt = jnp.zeros((EMBED_DIM, 128), jnp.float32).at[:, :2].set(W.T)
    bp = jnp.zeros((1, 128), jnp.float32).at[0, :2].set(b)
    y = pl.pallas_call(
        _tc_linear_body,
        out_shape=jax.ShapeDtypeStruct((PAD_B, 128), jnp.float32),
    )(rows, wt, bp)
    return y[:SEQ, :2]


# speedup vs baseline: 19.3567x; 11.3530x over previous
import functools

import jax
import jax.numpy as jnp
from jax import lax
from jax.experimental import pallas as pl
from jax.experimental.pallas import tpu as pltpu
from jax.experimental.pallas import tpu_sc as plsc

SEQ = 200
EMBED_DIM = 64
PAD_B = 256

_info = plsc.get_sparse_core_info()
_NC, _NS = _info.num_cores, _info.num_subcores
_NW = _NC * _NS
_B_PER_W = PAD_B // _NW

_sc_mesh = plsc.VectorSubcoreMesh(core_axis_name="c", subcore_axis_name="s")


@functools.partial(
    pl.kernel,
    mesh=_sc_mesh,
    out_type=jax.ShapeDtypeStruct((PAD_B, EMBED_DIM), jnp.float32),
    scratch_types=[
        pltpu.VMEM((16,), jnp.int32),
        pltpu.VMEM((_B_PER_W, EMBED_DIM, 128), jnp.float32),
        pltpu.VMEM((_B_PER_W, EMBED_DIM), jnp.float32),
        pltpu.SemaphoreType.DMA,
    ],
    compiler_params=pltpu.CompilerParams(
        use_tc_tiling_on_sc=True, needs_layout_passes=False
    ),
)
def _sc_gather(table_t_hbm, idx_hbm, out_hbm, idx_v, stripes_v, rows_v, sem):
    wid = lax.axis_index("s") * _NC + lax.axis_index("c")
    base = wid * _B_PER_W
    pltpu.sync_copy(idx_hbm.at[pl.ds(base, _B_PER_W)], idx_v.at[pl.ds(0, _B_PER_W)])
    idx_vec = idx_v[...]
    copies = []
    for j in range(_B_PER_W):
        col0 = pl.multiple_of((idx_vec[j] // 128) * 128, 128)
        copies.append(
            pltpu.async_copy(
                table_t_hbm.at[:, pl.ds(col0, 128)], stripes_v.at[j], sem
            )
        )
    for cp in copies:
        cp.wait()
    lane = lax.iota(jnp.int32, 16)
    for j in range(_B_PER_W):
        col = jnp.full((16,), idx_vec[j] % 128, jnp.int32)
        for c in range(4):
            vals = plsc.load_gather(stripes_v.at[j], [lane + c * 16, col])
            rows_v[j, pl.ds(c * 16, 16)] = vals
    pltpu.sync_copy(rows_v, out_hbm.at[pl.ds(base, _B_PER_W)])


def _tc_linear_body(g_ref, wt_ref, b_ref, o_ref):
    o_ref[...] = (
        jnp.dot(g_ref[...], wt_ref[...], preferred_element_type=jnp.float32)
        + b_ref[...]
    )


def kernel(sentence, emb_table, W, b):
    idx = sentence[-1].astype(jnp.int32)
    idx_pad = jnp.zeros((PAD_B,), jnp.int32).at[:SEQ].set(idx)
    rows = _sc_gather(emb_table.T, idx_pad)
    wt = jnp.zeros((EMBED_DIM, 128), jnp.float32).at[:, :2].set(W.T)
    bp = jnp.zeros((1, 128), jnp.float32).at[0, :2].set(b)
    y = pl.pallas_call(
        _tc_linear_body,
        out_shape=jax.ShapeDtypeStruct((PAD_B, 128), jnp.float32),
    )(rows, wt, bp)
    return y[:SEQ, :2]


# speedup vs baseline: 23.1431x; 1.1956x over previous
import functools

import jax
import jax.numpy as jnp
from jax import lax
from jax.experimental import pallas as pl
from jax.experimental.pallas import tpu as pltpu
from jax.experimental.pallas import tpu_sc as plsc

SEQ = 200
EMBED_DIM = 64
PAD_B = 256

_info = plsc.get_sparse_core_info()
_NC, _NS = _info.num_cores, _info.num_subcores
_NW = _NC * _NS
_B_PER_W = PAD_B // _NW

_sc_mesh = plsc.VectorSubcoreMesh(core_axis_name="c", subcore_axis_name="s")


@functools.partial(
    pl.kernel,
    mesh=_sc_mesh,
    out_type=jax.ShapeDtypeStruct((PAD_B, 2), jnp.float32),
    scratch_types=[
        pltpu.VMEM((16,), jnp.int32),
        pltpu.VMEM((2, EMBED_DIM), jnp.float32),
        pltpu.VMEM((16,), jnp.float32),
        pltpu.VMEM((_B_PER_W, EMBED_DIM, 128), jnp.float32),
        pltpu.VMEM((_B_PER_W, 2), jnp.float32),
        pltpu.SemaphoreType.DMA,
    ],
    compiler_params=pltpu.CompilerParams(
        use_tc_tiling_on_sc=True, needs_layout_passes=False
    ),
)
def _sc_embed_linear(
    table_t_hbm, idx_hbm, w_hbm, b_hbm, out_hbm, idx_v, w_v, b_v, stripes_v, y_v, sem
):
    wid = lax.axis_index("s") * _NC + lax.axis_index("c")
    base = wid * _B_PER_W

    @pl.when(base < SEQ)
    def _():
        pltpu.sync_copy(
            idx_hbm.at[pl.ds(base, _B_PER_W)], idx_v.at[pl.ds(0, _B_PER_W)]
        )
        pltpu.sync_copy(w_hbm, w_v)
        pltpu.sync_copy(b_hbm, b_v.at[pl.ds(0, 2)])
        idx_vec = idx_v[...]
        copies = []
        for j in range(_B_PER_W):
            col0 = pl.multiple_of((idx_vec[j] // 128) * 128, 128)
            copies.append(
                pltpu.async_copy(
                    table_t_hbm.at[:, pl.ds(col0, 128)], stripes_v.at[j], sem
                )
            )
        for cp in copies:
            cp.wait()
        lane = lax.iota(jnp.int32, 16)
        row_mask = lane < _B_PER_W
        col_vec = idx_vec % 128
        zeros = jnp.zeros((16,), jnp.float32)
        acc0, acc1 = zeros, zeros
        b_vec = b_v[...]
        for c in range(4):
            w0c = w_v[0, pl.ds(c * 16, 16)]
            w1c = w_v[1, pl.ds(c * 16, 16)]
            for dd in range(16):
                d = c * 16 + dd
                vals = plsc.load_gather(
                    stripes_v,
                    [lane, jnp.full((16,), d, jnp.int32), col_vec],
                    mask=row_mask,
                )
                acc0 = acc0 + vals * w0c[dd]
                acc1 = acc1 + vals * w1c[dd]
        acc0 = acc0 + b_vec[0]
        acc1 = acc1 + b_vec[1]
        col0i = jnp.zeros((16,), jnp.int32)
        plsc.store_scatter(y_v, [lane, col0i], acc0, mask=row_mask)
        plsc.store_scatter(y_v, [lane, col0i + 1], acc1, mask=row_mask)
        pltpu.sync_copy(y_v, out_hbm.at[pl.ds(base, _B_PER_W)])


def kernel(sentence, emb_table, W, b):
    idx = sentence[-1].astype(jnp.int32)
    y = _sc_embed_linear(emb_table.T, idx, W, b)
    return y[:SEQ]


# speedup vs baseline: 24.0440x; 1.0389x over previous
import functools

import jax
import jax.numpy as jnp
from jax import lax
from jax.experimental import pallas as pl
from jax.experimental.pallas import tpu as pltpu
from jax.experimental.pallas import tpu_sc as plsc

SEQ = 200
EMBED_DIM = 64
PAD_B = 256

_info = plsc.get_sparse_core_info()
_NC, _NS = _info.num_cores, _info.num_subcores
_NW = _NC * _NS
_B_PER_W = PAD_B // _NW

_sc_mesh = plsc.VectorSubcoreMesh(core_axis_name="c", subcore_axis_name="s")


@functools.partial(
    pl.kernel,
    mesh=_sc_mesh,
    out_type=jax.ShapeDtypeStruct((PAD_B, 2), jnp.float32),
    scratch_types=[
        pltpu.VMEM((16,), jnp.int32),
        pltpu.VMEM((2, EMBED_DIM), jnp.float32),
        pltpu.VMEM((16,), jnp.float32),
        pltpu.VMEM((_B_PER_W, EMBED_DIM, 128), jnp.float32),
        pltpu.VMEM((_B_PER_W, 2), jnp.float32),
        pltpu.SemaphoreType.DMA,
        pltpu.SemaphoreType.DMA,
    ],
    compiler_params=pltpu.CompilerParams(
        use_tc_tiling_on_sc=True, needs_layout_passes=False
    ),
)
def _sc_embed_linear(
    table_t_hbm,
    idx_hbm,
    w_hbm,
    b_hbm,
    out_hbm,
    idx_v,
    w_v,
    b_v,
    stripes_v,
    y_v,
    sem,
    sem2,
):
    wid = lax.axis_index("s") * _NC + lax.axis_index("c")
    base = wid * _B_PER_W

    @pl.when(base < SEQ)
    def _():
        idx_cp = pltpu.async_copy(
            idx_hbm.at[pl.ds(base, _B_PER_W)], idx_v.at[pl.ds(0, _B_PER_W)], sem2
        )
        w_cp = pltpu.async_copy(w_hbm, w_v, sem2)
        b_cp = pltpu.async_copy(b_hbm, b_v.at[pl.ds(0, 2)], sem2)
        idx_cp.wait()
        idx_vec = idx_v[...]
        copies = []
        for j in range(_B_PER_W):
            col0 = pl.multiple_of((idx_vec[j] // 128) * 128, 128)
            copies.append(
                pltpu.async_copy(
                    table_t_hbm.at[:, pl.ds(col0, 128)], stripes_v.at[j], sem
                )
            )
        w_cp.wait()
        b_cp.wait()
        for cp in copies:
            cp.wait()
        lane = lax.iota(jnp.int32, 16)
        row_mask = lane < _B_PER_W
        col_vec = idx_vec % 128
        zeros = jnp.zeros((16,), jnp.float32)
        acc0, acc1 = zeros, zeros
        b_vec = b_v[...]
        for c in range(4):
            w0c = w_v[0, pl.ds(c * 16, 16)]
            w1c = w_v[1, pl.ds(c * 16, 16)]
            for dd in range(16):
                d = c * 16 + dd
                vals = plsc.load_gather(
                    stripes_v,
                    [lane, jnp.full((16,), d, jnp.int32), col_vec],
                    mask=row_mask,
                )
                acc0 = acc0 + vals * w0c[dd]
                acc1 = acc1 + vals * w1c[dd]
        acc0 = acc0 + b_vec[0]
        acc1 = acc1 + b_vec[1]
        col0i = jnp.zeros((16,), jnp.int32)
        plsc.store_scatter(y_v, [lane, col0i], acc0, mask=row_mask)
        plsc.store_scatter(y_v, [lane, col0i + 1], acc1, mask=row_mask)
        pltpu.sync_copy(y_v, out_hbm.at[pl.ds(base, _B_PER_W)])


def kernel(sentence, emb_table, W, b):
    idx = sentence[-1].astype(jnp.int32)
    y = _sc_embed_linear(emb_table.T, idx, W, b)
    return y[:SEQ]


# speedup vs baseline: 24.1462x; 1.0042x over previous
import functools

import jax
import jax.numpy as jnp
from jax import lax
from jax.experimental import pallas as pl
from jax.experimental.pallas import tpu as pltpu
from jax.experimental.pallas import tpu_sc as plsc

SEQ = 200
EMBED_DIM = 64
BATCH = 4096

_info = plsc.get_sparse_core_info()
_NC, _NS = _info.num_cores, _info.num_subcores
_NW = _NC * _NS
_B_PER_W = 8

_sc_mesh = plsc.VectorSubcoreMesh(core_axis_name="c", subcore_axis_name="s")


@functools.partial(
    pl.kernel,
    mesh=_sc_mesh,
    out_type=jax.ShapeDtypeStruct((SEQ, 2), jnp.float32),
    scratch_types=[
        pltpu.VMEM((_B_PER_W, 128), jnp.int32),
        pltpu.VMEM((2, EMBED_DIM), jnp.float32),
        pltpu.VMEM((16,), jnp.float32),
        pltpu.VMEM((_B_PER_W, EMBED_DIM, 128), jnp.float32),
        pltpu.VMEM((_B_PER_W, 2), jnp.float32),
        pltpu.SemaphoreType.DMA,
        pltpu.SemaphoreType.DMA,
    ],
    compiler_params=pltpu.CompilerParams(
        use_tc_tiling_on_sc=True, needs_layout_passes=False
    ),
)
def _sc_embed_linear(
    sent_t_hbm,
    table_t_hbm,
    w_hbm,
    b_hbm,
    out_hbm,
    sent_v,
    w_v,
    b_v,
    stripes_v,
    y_v,
    sem,
    sem2,
):
    wid = lax.axis_index("s") * _NC + lax.axis_index("c")
    base = wid * _B_PER_W

    @pl.when(base < SEQ)
    def _():
        col_blk = (BATCH // 128 - 1) * 128
        s_cp = pltpu.async_copy(
            sent_t_hbm.at[pl.ds(base, _B_PER_W), pl.ds(col_blk, 128)], sent_v, sem2
        )
        w_cp = pltpu.async_copy(w_hbm, w_v, sem2)
        b_cp = pltpu.async_copy(b_hbm, b_v.at[pl.ds(0, 2)], sem2)
        lane = lax.iota(jnp.int32, 16)
        row_mask = lane < _B_PER_W
        s_cp.wait()
        idx_vec = plsc.load_gather(
            sent_v, [lane, jnp.full((16,), 127, jnp.int32)], mask=row_mask
        )
        copies = []
        for j in range(_B_PER_W):
            col0 = pl.multiple_of((idx_vec[j] // 128) * 128, 128)
            copies.append(
                pltpu.async_copy(
                    table_t_hbm.at[:, pl.ds(col0, 128)], stripes_v.at[j], sem
                )
            )
        w_cp.wait()
        b_cp.wait()
        for cp in copies:
            cp.wait()
        col_vec = idx_vec % 128
        zeros = jnp.zeros((16,), jnp.float32)
        acc0, acc1 = zeros, zeros
        b_vec = b_v[...]
        for c in range(4):
            w0c = w_v[0, pl.ds(c * 16, 16)]
            w1c = w_v[1, pl.ds(c * 16, 16)]
            for dd in range(16):
                d = c * 16 + dd
                vals = plsc.load_gather(
                    stripes_v,
                    [lane, jnp.full((16,), d, jnp.int32), col_vec],
                    mask=row_mask,
                )
                acc0 = acc0 + vals * w0c[dd]
                acc1 = acc1 + vals * w1c[dd]
        acc0 = acc0 + b_vec[0]
        acc1 = acc1 + b_vec[1]
        col0i = jnp.zeros((16,), jnp.int32)
        plsc.store_scatter(y_v, [lane, col0i], acc0, mask=row_mask)
        plsc.store_scatter(y_v, [lane, col0i + 1], acc1, mask=row_mask)
        pltpu.sync_copy(y_v, out_hbm.at[pl.ds(base, _B_PER_W)])


def kernel(sentence, emb_table, W, b):
    return _sc_embed_linear(sentence.T.astype(jnp.int32), emb_table.T, W, b)


# speedup vs baseline: 24.3232x; 1.0073x over previous
import functools

import jax
import jax.numpy as jnp
from jax import lax
from jax.experimental import pallas as pl
from jax.experimental.pallas import tpu as pltpu
from jax.experimental.pallas import tpu_sc as plsc

SEQ = 200
EMBED_DIM = 64
BATCH = 4096

_info = plsc.get_sparse_core_info()
_NC, _NS = _info.num_cores, _info.num_subcores
_NW = _NC * _NS
_B_PER_W = 8

_sc_mesh = plsc.VectorSubcoreMesh(core_axis_name="c", subcore_axis_name="s")


@functools.partial(
    pl.kernel,
    mesh=_sc_mesh,
    out_type=jax.ShapeDtypeStruct((SEQ, 2), jnp.float32),
    scratch_types=[
        pltpu.VMEM((_B_PER_W, 128), jnp.int32),
        pltpu.VMEM((2, EMBED_DIM), jnp.float32),
        pltpu.VMEM((16,), jnp.float32),
        pltpu.VMEM((_B_PER_W, EMBED_DIM, 128), jnp.float32),
        pltpu.VMEM((_B_PER_W, 2), jnp.float32),
        pltpu.SemaphoreType.DMA,
        pltpu.SemaphoreType.DMA,
    ],
    compiler_params=pltpu.CompilerParams(
        use_tc_tiling_on_sc=True, needs_layout_passes=False
    ),
)
def _sc_embed_linear(
    sent_t_hbm,
    table_t_hbm,
    w_hbm,
    b_hbm,
    out_hbm,
    sent_v,
    w_v,
    b_v,
    stripes_v,
    y_v,
    sem,
    sem2,
):
    wid = lax.axis_index("s") * _NC + lax.axis_index("c")
    base = wid * _B_PER_W

    @pl.when(base < SEQ)
    def _():
        col_blk = (BATCH // 128 - 1) * 128
        s_cp = pltpu.async_copy(
            sent_t_hbm.at[pl.ds(base, _B_PER_W), pl.ds(col_blk, 128)], sent_v, sem2
        )
        w_cp = pltpu.async_copy(w_hbm, w_v, sem2)
        b_cp = pltpu.async_copy(b_hbm, b_v.at[pl.ds(0, 2)], sem2)
        lane = lax.iota(jnp.int32, 16)
        row_mask = lane < _B_PER_W
        s_cp.wait()
        idx_vec = plsc.load_gather(
            sent_v, [lane, jnp.full((16,), 127, jnp.int32)], mask=row_mask
        )
        copies = []
        for j in range(_B_PER_W):
            col0 = pl.multiple_of((idx_vec[j] // 128) * 128, 128)
            copies.append(
                pltpu.async_copy(
                    table_t_hbm.at[:, pl.ds(col0, 128)], stripes_v.at[j], sem
                )
            )
        w_cp.wait()
        b_cp.wait()
        for cp in copies:
            cp.wait()
        col_vec = idx_vec % 128
        zeros = jnp.zeros((16,), jnp.float32)
        zeros_i = jnp.zeros((16,), jnp.int32)
        ones_i = zeros_i + 1
        b_vec = b_v[...]

        def body(d, carry):
            a0, a1 = carry
            dvec = jnp.full((16,), d, jnp.int32)
            vals = plsc.load_gather(
                stripes_v, [lane, dvec, col_vec], mask=row_mask
            )
            w0v = plsc.load_gather(w_v, [zeros_i, dvec])
            w1v = plsc.load_gather(w_v, [ones_i, dvec])
            return a0 + vals * w0v, a1 + vals * w1v

        acc0, acc1 = lax.fori_loop(0, EMBED_DIM, body, (zeros, zeros))
        acc0 = acc0 + b_vec[0]
        acc1 = acc1 + b_vec[1]
        col0i = jnp.zeros((16,), jnp.int32)
        plsc.store_scatter(y_v, [lane, col0i], acc0, mask=row_mask)
        plsc.store_scatter(y_v, [lane, col0i + 1], acc1, mask=row_mask)
        pltpu.sync_copy(y_v, out_hbm.at[pl.ds(base, _B_PER_W)])


def kernel(sentence, emb_table, W, b):
    return _sc_embed_linear(sentence.T.astype(jnp.int32), emb_table.T, W, b)


# speedup vs baseline: 24.4279x; 1.0043x over previous
import functools

import jax
import jax.numpy as jnp
from jax import lax
from jax.experimental import pallas as pl
from jax.experimental.pallas import tpu as pltpu
from jax.experimental.pallas import tpu_sc as plsc

SEQ = 200
EMBED_DIM = 64
BATCH = 4096

_info = plsc.get_sparse_core_info()
_NC, _NS = _info.num_cores, _info.num_subcores
_NW = _NC * _NS
_B_PER_W = 8

_sc_mesh = plsc.VectorSubcoreMesh(core_axis_name="c", subcore_axis_name="s")


@functools.partial(
    pl.kernel,
    mesh=_sc_mesh,
    out_type=jax.ShapeDtypeStruct((SEQ, 2), jnp.float32),
    scratch_types=[
        pltpu.VMEM((_B_PER_W, 128), jnp.int32),
        pltpu.VMEM((2, EMBED_DIM), jnp.float32),
        pltpu.VMEM((16,), jnp.float32),
        pltpu.VMEM((_B_PER_W, EMBED_DIM, 128), jnp.float32),
        pltpu.VMEM((_B_PER_W, 2), jnp.float32),
        pltpu.SemaphoreType.DMA,
        pltpu.SemaphoreType.DMA,
    ],
    compiler_params=pltpu.CompilerParams(
        use_tc_tiling_on_sc=True, needs_layout_passes=False, skip_device_barrier=True
    ),
)
def _sc_embed_linear(
    sent_t_hbm,
    table_t_hbm,
    w_hbm,
    b_hbm,
    out_hbm,
    sent_v,
    w_v,
    b_v,
    stripes_v,
    y_v,
    sem,
    sem2,
):
    wid = lax.axis_index("s") * _NC + lax.axis_index("c")
    base = wid * _B_PER_W

    @pl.when(base < SEQ)
    def _():
        col_blk = (BATCH // 128 - 1) * 128
        s_cp = pltpu.async_copy(
            sent_t_hbm.at[pl.ds(base, _B_PER_W), pl.ds(col_blk, 128)], sent_v, sem2
        )
        w_cp = pltpu.async_copy(w_hbm, w_v, sem2)
        b_cp = pltpu.async_copy(b_hbm, b_v.at[pl.ds(0, 2)], sem2)
        lane = lax.iota(jnp.int32, 16)
        row_mask = lane < _B_PER_W
        s_cp.wait()
        idx_vec = plsc.load_gather(
            sent_v, [lane, jnp.full((16,), 127, jnp.int32)], mask=row_mask
        )
        copies = []
        for j in range(_B_PER_W):
            col0 = pl.multiple_of((idx_vec[j] // 128) * 128, 128)
            copies.append(
                pltpu.async_copy(
                    table_t_hbm.at[:, pl.ds(col0, 128)], stripes_v.at[j], sem
                )
            )
        w_cp.wait()
        b_cp.wait()
        for cp in copies:
            cp.wait()
        col_vec = idx_vec % 128
        zeros = jnp.zeros((16,), jnp.float32)
        zeros_i = jnp.zeros((16,), jnp.int32)
        ones_i = zeros_i + 1
        b_vec = b_v[...]

        def body(d, carry):
            a0, a1 = carry
            dvec = jnp.full((16,), d, jnp.int32)
            vals = plsc.load_gather(
                stripes_v, [lane, dvec, col_vec], mask=row_mask
            )
            w0v = plsc.load_gather(w_v, [zeros_i, dvec])
            w1v = plsc.load_gather(w_v, [ones_i, dvec])
            return a0 + vals * w0v, a1 + vals * w1v

        acc0, acc1 = lax.fori_loop(0, EMBED_DIM, body, (zeros, zeros))
        acc0 = acc0 + b_vec[0]
        acc1 = acc1 + b_vec[1]
        col0i = jnp.zeros((16,), jnp.int32)
        plsc.store_scatter(y_v, [lane, col0i], acc0, mask=row_mask)
        plsc.store_scatter(y_v, [lane, col0i + 1], acc1, mask=row_mask)
        pltpu.sync_copy(y_v, out_hbm.at[pl.ds(base, _B_PER_W)])


def kernel(sentence, emb_table, W, b):
    return _sc_embed_linear(sentence.T.astype(jnp.int32), emb_table.T, W, b)
